# baseline (speedup 1.0000x reference)
import functools

import jax
import jax.numpy as jnp
from jax import lax
from jax.experimental import pallas as pl
from jax.experimental.pallas import tpu as pltpu
from jax.experimental.pallas import tpu_sc as plsc

NC = 2
NS = 16
NW = NC * NS
K = 125
ZR = 25
DA = 112
DB = 16


def _zero_vmem(ref, rows, width):
    z16 = jnp.zeros((16,), jnp.float32)

    def body(i, carry):
        for k in range(width // 16):
            ref[i, pl.ds(k * 16, 16)] = z16
        return carry

    lax.fori_loop(0, rows, body, 0)


def _make_deg(N, E):
    epw = E // NW
    nch = epw // K
    rpt = N // NS
    mesh = plsc.VectorSubcoreMesh(core_axis_name="c", subcore_axis_name="s")

    @functools.partial(
        pl.kernel,
        out_type=jax.ShapeDtypeStruct((NC, N, 16), jnp.float32),
        mesh=mesh,
        compiler_params=pltpu.CompilerParams(use_tc_tiling_on_sc=False),
        scratch_types=[
            pltpu.VMEM((nch, K), jnp.int32),
            pltpu.VMEM((K, 16), jnp.float32),
            pltpu.VMEM((ZR, 16), jnp.float32),
            pltpu.SemaphoreType.DMA,
            pltpu.VMEM_SHARED((N, 16), jnp.float32),
        ],
    )
    def deg(dst_hbm, out_hbm, dst_v, ones_v, zr_v, dsem, acc_sh):
        c = lax.axis_index("c")
        s = lax.axis_index("s")
        wid = c * NS + s

        pltpu.async_copy(dst_hbm.at[wid], dst_v, dsem)
        one16 = jnp.ones((16,), jnp.float32)

        def ones_body(i, carry):
            ones_v[i, pl.ds(0, 16)] = one16
            return carry

        lax.fori_loop(0, K, ones_body, 0)
        _zero_vmem(zr_v, ZR, 16)

        def zacc_body(j, carry):
            pltpu.sync_copy(zr_v, acc_sh.at[pl.ds(s * rpt + j * ZR, ZR)])
            return carry

        lax.fori_loop(0, rpt // ZR, zacc_body, 0)
        pltpu.make_async_copy(dst_hbm.at[wid], dst_v, dsem).wait()
        plsc.subcore_barrier()

        def step(j, carry):
            pltpu.sync_copy(ones_v, acc_sh.at[dst_v.at[j]], add=True)
            return carry

        lax.fori_loop(0, nch, step, 0)
        plsc.subcore_barrier()
        pltpu.sync_copy(acc_sh.at[pl.ds(s * rpt, rpt)],
                        out_hbm.at[c].at[pl.ds(s * rpt, rpt)])

    return deg


def _make_agg(N, E, G=20):
    epw = E // NW
    nch = epw // K
    rpt = N // NS
    assert nch % G == 0 and G % 2 == 0 and rpt % K == 0
    mesh = plsc.VectorSubcoreMesh(core_axis_name="c", subcore_axis_name="s")

    @functools.partial(
        pl.kernel,
        out_type=(jax.ShapeDtypeStruct((NC, N, DA), jnp.float32),
                  jax.ShapeDtypeStruct((NC, N, DB), jnp.float32)),
        mesh=mesh,
        compiler_params=pltpu.CompilerParams(use_tc_tiling_on_sc=False),
        scratch_types=[
            pltpu.VMEM((G, K), jnp.int32),
            pltpu.VMEM((G, K), jnp.int32),
            pltpu.VMEM((K, DA), jnp.float32),
            pltpu.VMEM((K, DA), jnp.float32),
            pltpu.VMEM((K, DB), jnp.float32),
            pltpu.VMEM((K, DB), jnp.float32),
            pltpu.SemaphoreType.DMA,
            pltpu.SemaphoreType.DMA,
            pltpu.SemaphoreType.DMA,
            pltpu.SemaphoreType.DMA,
            pltpu.VMEM_SHARED((N, DA), jnp.float32),
            pltpu.VMEM_SHARED((N, DB), jnp.float32),
            pltpu.VMEM_SHARED((N, DB), jnp.float32),
        ],
    )
    def agg(hA_hbm, hB_hbm, src_hbm, dst_hbm, outA_hbm, outB_hbm,
            src_v, dst_v, rowsA0_v, rowsA1_v, rowsB0_v, rowsB1_v,
            gsA0, gsA1, gsB0, gsB1, accA_sh, accB_sh, hB_sh):
        c = lax.axis_index("c")
        s = lax.axis_index("s")
        wid = c * NS + s

        pltpu.async_copy(src_hbm.at[wid].at[pl.ds(0, G)], src_v, gsA0)
        pltpu.async_copy(dst_hbm.at[wid].at[pl.ds(0, G)], dst_v, gsA1)
        pltpu.async_copy(hB_hbm.at[pl.ds(s * rpt, rpt)],
                         hB_sh.at[pl.ds(s * rpt, rpt)], gsB0)

        _zero_vmem(rowsA1_v, K, DA)
        _zero_vmem(rowsB1_v, K, DB)

        def zacc_body(j, carry):
            pltpu.sync_copy(rowsA1_v, accA_sh.at[pl.ds(s * rpt + j * K, K)])
            pltpu.sync_copy(rowsB1_v, accB_sh.at[pl.ds(s * rpt + j * K, K)])
            return carry

        lax.fori_loop(0, rpt // K, zacc_body, 0)
        pltpu.make_async_copy(src_hbm.at[wid].at[pl.ds(0, G)], src_v, gsA0).wait()
        pltpu.make_async_copy(dst_hbm.at[wid].at[pl.ds(0, G)], dst_v, gsA1).wait()
        pltpu.make_async_copy(hB_hbm.at[pl.ds(s * rpt, rpt)],
                              hB_sh.at[pl.ds(s * rpt, rpt)], gsB0).wait()
        pltpu.async_copy(hA_hbm.at[src_v.at[0]], rowsA0_v, gsA0)
        plsc.subcore_barrier()

        def group(g, carry):
            @pl.when(g > 0)
            def _():
                pltpu.sync_copy(src_hbm.at[wid].at[pl.ds(g * G, G)], src_v)
                pltpu.sync_copy(dst_hbm.at[wid].at[pl.ds(g * G, G)], dst_v)
                pltpu.async_copy(hA_hbm.at[src_v.at[0]], rowsA0_v, gsA0)

            pltpu.async_copy(hB_sh.at[src_v.at[0]], rowsB0_v, gsB0)

            def pair(jj, carry2):
                j0 = jj * 2
                j1 = j0 + 1
                pltpu.async_copy(hA_hbm.at[src_v.at[j1]], rowsA1_v, gsA1)
                pltpu.async_copy(hB_sh.at[src_v.at[j1]], rowsB1_v, gsB1)
                pltpu.make_async_copy(hA_hbm.at[src_v.at[j0]], rowsA0_v, gsA0).wait()
                pltpu.sync_copy(rowsA0_v, accA_sh.at[dst_v.at[j0]], add=True)
                pltpu.make_async_copy(hB_sh.at[src_v.at[j0]], rowsB0_v, gsB0).wait()
                pltpu.sync_copy(rowsB0_v, accB_sh.at[dst_v.at[j0]], add=True)

                @pl.when(jj < G // 2 - 1)
                def _():
                    pltpu.async_copy(hA_hbm.at[src_v.at[j0 + 2]], rowsA0_v, gsA0)
                    pltpu.async_copy(hB_sh.at[src_v.at[j0 + 2]], rowsB0_v, gsB0)

                pltpu.make_async_copy(hA_hbm.at[src_v.at[j1]], rowsA1_v, gsA1).wait()
                pltpu.sync_copy(rowsA1_v, accA_sh.at[dst_v.at[j1]], add=True)
                pltpu.make_async_copy(hB_sh.at[src_v.at[j1]], rowsB1_v, gsB1).wait()
                pltpu.sync_copy(rowsB1_v, accB_sh.at[dst_v.at[j1]], add=True)
                return carry2

            lax.fori_loop(0, G // 2, pair, 0)
            return carry

        lax.fori_loop(0, nch // G, group, 0)
        plsc.subcore_barrier()
        pltpu.sync_copy(accA_sh.at[pl.ds(s * rpt, rpt)],
                        outA_hbm.at[c].at[pl.ds(s * rpt, rpt)])
        pltpu.sync_copy(accB_sh.at[pl.ds(s * rpt, rpt)],
                        outB_hbm.at[c].at[pl.ds(s * rpt, rpt)])

    return agg


_BLK = 2000


def _cat(a, b):
    return jnp.concatenate([a, b], axis=1)


def _c1_body(pa_ref, pb_ref, pd_ref, x_ref, wl_ref, wr_ref, b_ref,
             ha_ref, hb_ref, r_ref):
    deg = jnp.maximum(pd_ref[0, :, 0:1] + pd_ref[1, :, 0:1], 1.0)
    r = 1.0 / deg
    agg = _cat(pa_ref[0] + pa_ref[1], pb_ref[0] + pb_ref[1]) * r
    h = (jnp.dot(agg, wl_ref[...], preferred_element_type=jnp.float32)
         + jnp.dot(x_ref[...], wr_ref[...], preferred_element_type=jnp.float32)
         + b_ref[...])
    h = jnp.maximum(h, 0.0)
    ha_ref[...] = h[:, :DA]
    hb_ref[...] = h[:, DA:]
    r_ref[...] = r


def _c2_body(pa_ref, pb_ref, ha_ref, hb_ref, r_ref, wl_ref, wr_ref, b_ref,
             oa_ref, ob_ref):
    agg = _cat(pa_ref[0] + pa_ref[1], pb_ref[0] + pb_ref[1]) * r_ref[...]
    h = (jnp.dot(agg, wl_ref[...], preferred_element_type=jnp.float32)
         + jnp.dot(_cat(ha_ref[...], hb_ref[...]), wr_ref[...],
                   preferred_element_type=jnp.float32)
         + b_ref[...])
    h = jnp.maximum(h, 0.0)
    oa_ref[...] = h[:, :DA]
    ob_ref[...] = h[:, DA:]


def _c3_body(pa_ref, pb_ref, ha_ref, hb_ref, r_ref, wl_ref, wr_ref, b_ref,
             wf_ref, bf_ref, o_ref):
    agg = _cat(pa_ref[0] + pa_ref[1], pb_ref[0] + pb_ref[1]) * r_ref[...]
    h = (jnp.dot(agg, wl_ref[...], preferred_element_type=jnp.float32)
         + jnp.dot(_cat(ha_ref[...], hb_ref[...]), wr_ref[...],
                   preferred_element_type=jnp.float32)
         + b_ref[...])
    o_ref[...] = jnp.dot(h, wf_ref[...], preferred_element_type=jnp.float32) + bf_ref[...]


def _full(shape):
    return pl.BlockSpec(shape, lambda i: (0,) * len(shape))


def _rows(w=128):
    return pl.BlockSpec((_BLK, w), lambda i: (i, 0))


def _rows3(w=128):
    return pl.BlockSpec((2, _BLK, w), lambda i: (0, i, 0))


def _combine1(pa, pb, pd, x, Wl, Wr, b, N):
    return pl.pallas_call(
        _c1_body,
        grid=(N // _BLK,),
        in_specs=[_rows3(DA), _rows3(DB), _rows3(16), _rows(), _full((128, 128)),
                  _full((128, 128)), _full((1, 128))],
        out_specs=[_rows(DA), _rows(DB), _rows(1)],
        out_shape=[jax.ShapeDtypeStruct((N, DA), jnp.float32),
                   jax.ShapeDtypeStruct((N, DB), jnp.float32),
                   jax.ShapeDtypeStruct((N, 1), jnp.float32)],
    )(pa, pb, pd, x, Wl, Wr, b)


def _combine2(pa, pb, ha, hb, r, Wl, Wr, b, N):
    return pl.pallas_call(
        _c2_body,
        grid=(N // _BLK,),
        in_specs=[_rows3(DA), _rows3(DB), _rows(DA), _rows(DB), _rows(1),
                  _full((128, 128)), _full((128, 128)), _full((1, 128))],
        out_specs=[_rows(DA), _rows(DB)],
        out_shape=[jax.ShapeDtypeStruct((N, DA), jnp.float32),
                   jax.ShapeDtypeStruct((N, DB), jnp.float32)],
    )(pa, pb, ha, hb, r, Wl, Wr, b)


def _combine3(pa, pb, ha, hb, r, Wl, Wr, b, Wfc, bfc, N, C):
    return pl.pallas_call(
        _c3_body,
        grid=(N // _BLK,),
        in_specs=[_rows3(DA), _rows3(DB), _rows(DA), _rows(DB), _rows(1),
                  _full((128, 128)), _full((128, 128)), _full((1, 128)),
                  _full((128, C)), _full((1, C))],
        out_specs=_rows(C),
        out_shape=jax.ShapeDtypeStruct((N, C), jnp.float32),
    )(pa, pb, ha, hb, r, Wl, Wr, b, Wfc, bfc)


def kernel(x, edge_index, W1l, b1l, W1r, W2l, b2l, W2r, W3l, b3l, W3r, Wfc, bfc):
    N, D = x.shape
    E = edge_index.shape[1]
    C = Wfc.shape[1]
    nch = (E // NW) // K

    src = edge_index[0].reshape(NW, nch, K)
    dst = edge_index[1].reshape(NW, nch, K)
    xa = x[:, :DA]
    xb = x[:, DA:]

    pd = _make_deg(N, E)(dst)
    agg = _make_agg(N, E)

    p1a, p1b = agg(xa, xb, src, dst)
    h1a, h1b, recip = _combine1(p1a, p1b, pd, x, W1l, W1r, b1l.reshape(1, -1), N)
    p2a, p2b = agg(h1a, h1b, src, dst)
    h2a, h2b = _combine2(p2a, p2b, h1a, h1b, recip, W2l, W2r,
                         b2l.reshape(1, -1), N)
    p3a, p3b = agg(h2a, h2b, src, dst)
    return _combine3(p3a, p3b, h2a, h2b, recip, W3l, W3r, b3l.reshape(1, -1),
                     Wfc, bfc.reshape(1, -1), N, C)


# speedup vs baseline: 1.3374x; 1.3374x over previous
import functools

import jax
import jax.numpy as jnp
from jax import lax
from jax.experimental import pallas as pl
from jax.experimental.pallas import tpu as pltpu
from jax.experimental.pallas import tpu_sc as plsc

NC = 2
NS = 16
NW = NC * NS
K = 125
ZR = 25


def _zero_vmem(ref, rows, width):
    z16 = jnp.zeros((16,), jnp.float32)

    def body(i, carry):
        for k in range(width // 16):
            ref[i, pl.ds(k * 16, 16)] = z16
        return carry

    lax.fori_loop(0, rows, body, 0)


def _make_deg(N, E):
    epw = E // NW
    nch = epw // K
    rpt = N // NS
    mesh = plsc.VectorSubcoreMesh(core_axis_name="c", subcore_axis_name="s")

    @functools.partial(
        pl.kernel,
        out_type=jax.ShapeDtypeStruct((NC, N, 16), jnp.float32),
        mesh=mesh,
        compiler_params=pltpu.CompilerParams(use_tc_tiling_on_sc=False),
        scratch_types=[
            pltpu.VMEM((nch, K), jnp.int32),
            pltpu.VMEM((K, 16), jnp.float32),
            pltpu.VMEM((ZR, 16), jnp.float32),
            pltpu.SemaphoreType.DMA,
            pltpu.VMEM_SHARED((N, 16), jnp.float32),
        ],
    )
    def deg(dst_hbm, out_hbm, dst_v, ones_v, zr_v, dsem, acc_sh):
        c = lax.axis_index("c")
        s = lax.axis_index("s")
        wid = c * NS + s

        pltpu.async_copy(dst_hbm.at[wid], dst_v, dsem)
        one16 = jnp.ones((16,), jnp.float32)

        def ones_body(i, carry):
            ones_v[i, pl.ds(0, 16)] = one16
            return carry

        lax.fori_loop(0, K, ones_body, 0)
        _zero_vmem(zr_v, ZR, 16)

        def zacc_body(j, carry):
            pltpu.sync_copy(zr_v, acc_sh.at[pl.ds(s * rpt + j * ZR, ZR)])
            return carry

        lax.fori_loop(0, rpt // ZR, zacc_body, 0)
        pltpu.make_async_copy(dst_hbm.at[wid], dst_v, dsem).wait()
        plsc.subcore_barrier()

        def step(j, carry):
            pltpu.sync_copy(ones_v, acc_sh.at[dst_v.at[j]], add=True)
            return carry

        lax.fori_loop(0, nch, step, 0)
        plsc.subcore_barrier()
        pltpu.sync_copy(acc_sh.at[pl.ds(s * rpt, rpt)],
                        out_hbm.at[c].at[pl.ds(s * rpt, rpt)])

    return deg


def _make_agg(N, E, D, G=20):
    epw = E // NW
    nch = epw // K
    rpt = N // NS
    ngr = nch // G
    assert nch % G == 0 and G % 2 == 0 and rpt % K == 0
    mesh = plsc.VectorSubcoreMesh(core_axis_name="c", subcore_axis_name="s")

    @functools.partial(
        pl.kernel,
        out_type=jax.ShapeDtypeStruct((NC, N, D), jnp.float32),
        mesh=mesh,
        compiler_params=pltpu.CompilerParams(use_tc_tiling_on_sc=False),
        scratch_types=[
            pltpu.VMEM((2, G, K), jnp.int32),
            pltpu.VMEM((2, G, K), jnp.int32),
            pltpu.VMEM((K, D), jnp.float32),
            pltpu.VMEM((K, D), jnp.float32),
            pltpu.SemaphoreType.DMA,
            pltpu.SemaphoreType.DMA,
            pltpu.SemaphoreType.DMA,
            pltpu.SemaphoreType.DMA,
            pltpu.VMEM_SHARED((N, D), jnp.float32),
        ],
    )
    def agg(h_hbm, src_hbm, dst_hbm, out_hbm, src_v, dst_v, rows0_v, rows1_v,
            gsem0, gsem1, isem0, isem1, acc_sh):
        c = lax.axis_index("c")
        s = lax.axis_index("s")
        wid = c * NS + s

        pltpu.async_copy(src_hbm.at[wid].at[pl.ds(0, G)], src_v.at[0], isem0)
        pltpu.async_copy(dst_hbm.at[wid].at[pl.ds(0, G)], dst_v.at[0], isem1)

        _zero_vmem(rows1_v, K, D)

        def zacc_body(j, carry):
            pltpu.sync_copy(rows1_v, acc_sh.at[pl.ds(s * rpt + j * K, K)])
            return carry

        lax.fori_loop(0, rpt // K, zacc_body, 0)

        pltpu.make_async_copy(src_hbm.at[wid].at[pl.ds(0, G)], src_v.at[0], isem0).wait()
        pltpu.make_async_copy(dst_hbm.at[wid].at[pl.ds(0, G)], dst_v.at[0], isem1).wait()
        pltpu.async_copy(h_hbm.at[src_v.at[0].at[0]], rows0_v, gsem0)
        plsc.subcore_barrier()

        def group(g, carry):
            p = lax.rem(g, 2)
            q = 1 - p

            @pl.when(g + 1 < ngr)
            def _():
                pltpu.async_copy(src_hbm.at[wid].at[pl.ds((g + 1) * G, G)],
                                 src_v.at[q], isem0)
                pltpu.async_copy(dst_hbm.at[wid].at[pl.ds((g + 1) * G, G)],
                                 dst_v.at[q], isem1)

            def pair(jj, carry2):
                j0 = jj * 2
                j1 = j0 + 1
                pltpu.async_copy(h_hbm.at[src_v.at[p].at[j1]], rows1_v, gsem1)
                pltpu.make_async_copy(h_hbm.at[src_v.at[p].at[j0]], rows0_v,
                                      gsem0).wait()
                pltpu.sync_copy(rows0_v, acc_sh.at[dst_v.at[p].at[j0]], add=True)

                @pl.when(jj < G // 2 - 1)
                def _():
                    pltpu.async_copy(h_hbm.at[src_v.at[p].at[j0 + 2]], rows0_v,
                                     gsem0)

                @pl.when(jnp.logical_and(jj == G // 2 - 1, g + 1 < ngr))
                def _():
                    pltpu.make_async_copy(
                        src_hbm.at[wid].at[pl.ds((g + 1) * G, G)], src_v.at[q],
                        isem0).wait()
                    pltpu.make_async_copy(
                        dst_hbm.at[wid].at[pl.ds((g + 1) * G, G)], dst_v.at[q],
                        isem1).wait()
                    pltpu.async_copy(h_hbm.at[src_v.at[q].at[0]], rows0_v, gsem0)

                pltpu.make_async_copy(h_hbm.at[src_v.at[p].at[j1]], rows1_v,
                                      gsem1).wait()
                pltpu.sync_copy(rows1_v, acc_sh.at[dst_v.at[p].at[j1]], add=True)
                return carry2

            lax.fori_loop(0, G // 2, pair, 0)
            return carry

        lax.fori_loop(0, ngr, group, 0)
        plsc.subcore_barrier()
        pltpu.sync_copy(acc_sh.at[pl.ds(s * rpt, rpt)],
                        out_hbm.at[c].at[pl.ds(s * rpt, rpt)])

    return agg


_BLK = 2000


def _c1_body(p_ref, pd_ref, x_ref, wl_ref, wr_ref, b_ref, h_ref, r_ref):
    deg = jnp.maximum(pd_ref[0, :, 0:1] + pd_ref[1, :, 0:1], 1.0)
    r = 1.0 / deg
    agg = (p_ref[0] + p_ref[1]) * r
    h = (jnp.dot(agg, wl_ref[...], preferred_element_type=jnp.float32)
         + jnp.dot(x_ref[...], wr_ref[...], preferred_element_type=jnp.float32)
         + b_ref[...])
    h_ref[...] = jnp.maximum(h, 0.0)
    r_ref[...] = r


def _c2_body(p_ref, h_ref, r_ref, wl_ref, wr_ref, b_ref, o_ref):
    agg = (p_ref[0] + p_ref[1]) * r_ref[...]
    h = (jnp.dot(agg, wl_ref[...], preferred_element_type=jnp.float32)
         + jnp.dot(h_ref[...], wr_ref[...], preferred_element_type=jnp.float32)
         + b_ref[...])
    o_ref[...] = jnp.maximum(h, 0.0)


def _c3_body(p_ref, h_ref, r_ref, wl_ref, wr_ref, b_ref, wf_ref, bf_ref, o_ref):
    agg = (p_ref[0] + p_ref[1]) * r_ref[...]
    h = (jnp.dot(agg, wl_ref[...], preferred_element_type=jnp.float32)
         + jnp.dot(h_ref[...], wr_ref[...], preferred_element_type=jnp.float32)
         + b_ref[...])
    o_ref[...] = jnp.dot(h, wf_ref[...], preferred_element_type=jnp.float32) + bf_ref[...]


def _full(shape):
    return pl.BlockSpec(shape, lambda i: (0,) * len(shape))


def _rows(w=128):
    return pl.BlockSpec((_BLK, w), lambda i: (i, 0))


def _rows3(w=128):
    return pl.BlockSpec((2, _BLK, w), lambda i: (0, i, 0))


def _combine1(p, pd, x, Wl, Wr, b, N):
    return pl.pallas_call(
        _c1_body,
        grid=(N // _BLK,),
        in_specs=[_rows3(), _rows3(16), _rows(), _full((128, 128)),
                  _full((128, 128)), _full((1, 128))],
        out_specs=[_rows(), _rows(1)],
        out_shape=[jax.ShapeDtypeStruct((N, 128), jnp.float32),
                   jax.ShapeDtypeStruct((N, 1), jnp.float32)],
    )(p, pd, x, Wl, Wr, b)


def _combine2(p, h, r, Wl, Wr, b, N):
    return pl.pallas_call(
        _c2_body,
        grid=(N // _BLK,),
        in_specs=[_rows3(), _rows(), _rows(1),
                  _full((128, 128)), _full((128, 128)), _full((1, 128))],
        out_specs=_rows(),
        out_shape=jax.ShapeDtypeStruct((N, 128), jnp.float32),
    )(p, h, r, Wl, Wr, b)


def _combine3(p, h, r, Wl, Wr, b, Wfc, bfc, N, C):
    return pl.pallas_call(
        _c3_body,
        grid=(N // _BLK,),
        in_specs=[_rows3(), _rows(), _rows(1),
                  _full((128, 128)), _full((128, 128)), _full((1, 128)),
                  _full((128, C)), _full((1, C))],
        out_specs=_rows(C),
        out_shape=jax.ShapeDtypeStruct((N, C), jnp.float32),
    )(p, h, r, Wl, Wr, b, Wfc, bfc)


def kernel(x, edge_index, W1l, b1l, W1r, W2l, b2l, W2r, W3l, b3l, W3r, Wfc, bfc):
    N, D = x.shape
    E = edge_index.shape[1]
    C = Wfc.shape[1]
    nch = (E // NW) // K

    src = edge_index[0].reshape(NW, nch, K)
    dst = edge_index[1].reshape(NW, nch, K)

    pd = _make_deg(N, E)(dst)
    agg = _make_agg(N, E, D)

    p1 = agg(x, src, dst)
    h1, recip = _combine1(p1, pd, x, W1l, W1r, b1l.reshape(1, -1), N)
    p2 = agg(h1, src, dst)
    h2 = _combine2(p2, h1, recip, W2l, W2r, b2l.reshape(1, -1), N)
    p3 = agg(h2, src, dst)
    return _combine3(p3, h2, recip, W3l, W3r, b3l.reshape(1, -1), Wfc,
                     bfc.reshape(1, -1), N, C)
